# baseline (speedup 1.0000x reference)
import jax
import jax.numpy as jnp
from jax import lax
from jax.experimental import pallas as pl
from jax.experimental.pallas import tpu as pltpu

_CHUNK = 2048


def _tc_gather(B, D):
  n_chunks = B // _CHUNK

  def body(idx_hbm, table_hbm, out_hbm, vmem, smem_idx, sem_i, sem_r):
    def chunk(c, _):
      pltpu.make_async_copy(idx_hbm.at[pl.ds(c * _CHUNK, _CHUNK)], smem_idx,
                            sem_i).start()
      pltpu.make_async_copy(idx_hbm.at[pl.ds(c * _CHUNK, _CHUNK)], smem_idx,
                            sem_i).wait()

      def row(i, _):
        r = smem_idx[i]
        pltpu.make_async_copy(table_hbm.at[r], vmem.at[c * _CHUNK + i],
                              sem_r).start()
        return 0

      lax.fori_loop(0, _CHUNK, row, 0)
      return 0

    lax.fori_loop(0, n_chunks, chunk, 0)

    def drain(i, _):
      pltpu.make_async_copy(table_hbm.at[0], vmem.at[0], sem_r).wait()
      return 0

    lax.fori_loop(0, B, drain, 0)
    pltpu.make_async_copy(vmem, out_hbm, sem_i).start()
    pltpu.make_async_copy(vmem, out_hbm, sem_i).wait()

  return pl.pallas_call(
      body,
      out_shape=jax.ShapeDtypeStruct((B, D), jnp.float32),
      in_specs=[
          pl.BlockSpec(memory_space=pl.ANY),
          pl.BlockSpec(memory_space=pl.ANY),
      ],
      out_specs=pl.BlockSpec(memory_space=pl.ANY),
      scratch_shapes=[
          pltpu.VMEM((B, D), jnp.float32),
          pltpu.SMEM((_CHUNK,), jnp.int32),
          pltpu.SemaphoreType.DMA,
          pltpu.SemaphoreType.DMA,
      ],
  )


@jax.jit
def kernel(users, U_g):
  flat = users.reshape(-1).astype(jnp.int32)
  out = _tc_gather(flat.shape[0], U_g.shape[1])(flat, U_g)
  return out.reshape(tuple(users.shape) + (U_g.shape[1],))


# speedup vs baseline: 1.2403x; 1.2403x over previous
import functools

import jax
import jax.numpy as jnp
from jax import lax
from jax.experimental import pallas as pl
from jax.experimental.pallas import tpu as pltpu, tpu_sc as plsc

_info = plsc.get_sparse_core_info()
_NC = _info.num_cores
_NS = _info.num_subcores
_NW = _NC * _NS

_G = 16
_TC_CHUNK = 1024


def _sc_gather(B, D):
  b_per_w = B // _NW
  n_groups = b_per_w // _G
  mesh = plsc.VectorSubcoreMesh(core_axis_name="c", subcore_axis_name="s")

  @functools.partial(
      pl.kernel,
      mesh=mesh,
      out_type=jax.ShapeDtypeStruct((B, D), jnp.float32),
      scratch_types=[
          pltpu.VMEM((b_per_w,), jnp.int32),
          pltpu.VMEM((b_per_w, D), jnp.float32),
          pltpu.SemaphoreType.DMA,
      ],
  )
  def gather_kernel(table_hbm, idx_hbm, out_hbm, idx_v, rows_v, sem):
    wid = lax.axis_index("s") * _NC + lax.axis_index("c")
    base = wid * b_per_w
    pltpu.sync_copy(idx_hbm.at[pl.ds(base, b_per_w)], idx_v)

    def fire(g, _):
      vec = idx_v[pl.ds(g * _G, _G)]
      for k in range(_G):
        pltpu.async_copy(table_hbm.at[vec[k]], rows_v.at[g * _G + k], sem)
      return 0

    lax.fori_loop(0, n_groups, fire, 0)

    def drain(g, _):
      for k in range(_G):
        pltpu.make_async_copy(table_hbm.at[0], rows_v.at[0], sem).wait()
      return 0

    lax.fori_loop(0, n_groups, drain, 0)
    pltpu.sync_copy(rows_v, out_hbm.at[pl.ds(base, b_per_w)])

  return gather_kernel


def _tc_gather(B, D):
  n_chunks = B // _TC_CHUNK

  def body(idx_hbm, table_hbm, out_hbm, vmem, smem_idx, sem_i, sem_r):
    def chunk(c, _):
      pltpu.make_async_copy(idx_hbm.at[pl.ds(c * _TC_CHUNK, _TC_CHUNK)],
                            smem_idx, sem_i).start()
      pltpu.make_async_copy(idx_hbm.at[pl.ds(c * _TC_CHUNK, _TC_CHUNK)],
                            smem_idx, sem_i).wait()

      def row(i, _):
        r = smem_idx[i]
        pltpu.make_async_copy(table_hbm.at[r], vmem.at[c * _TC_CHUNK + i],
                              sem_r).start()
        return 0

      lax.fori_loop(0, _TC_CHUNK, row, 0)
      return 0

    lax.fori_loop(0, n_chunks, chunk, 0)

    def drain(i, _):
      pltpu.make_async_copy(table_hbm.at[0], vmem.at[0], sem_r).wait()
      return 0

    lax.fori_loop(0, B, drain, 0)
    pltpu.make_async_copy(vmem, out_hbm, sem_i).start()
    pltpu.make_async_copy(vmem, out_hbm, sem_i).wait()

  return pl.pallas_call(
      body,
      out_shape=jax.ShapeDtypeStruct((B, D), jnp.float32),
      in_specs=[
          pl.BlockSpec(memory_space=pl.ANY),
          pl.BlockSpec(memory_space=pl.ANY),
      ],
      out_specs=pl.BlockSpec(memory_space=pl.ANY),
      scratch_shapes=[
          pltpu.VMEM((B, D), jnp.float32),
          pltpu.SMEM((_TC_CHUNK,), jnp.int32),
          pltpu.SemaphoreType.DMA,
          pltpu.SemaphoreType.DMA,
      ],
  )


@jax.jit
def kernel(users, U_g):
  flat = users.reshape(-1).astype(jnp.int32)
  B, D = flat.shape[0], U_g.shape[1]
  b_sc = 10240
  out_sc = _sc_gather(b_sc, D)(U_g, flat[:b_sc])
  out_tc = _tc_gather(B - b_sc, D)(flat[b_sc:], U_g)
  out = jnp.concatenate([out_sc, out_tc], axis=0)
  return out.reshape(tuple(users.shape) + (D,))


# speedup vs baseline: 1.2418x; 1.0012x over previous
import functools

import jax
import jax.numpy as jnp
from jax import lax
from jax.experimental import pallas as pl
from jax.experimental.pallas import tpu as pltpu, tpu_sc as plsc

_info = plsc.get_sparse_core_info()
_NC = _info.num_cores
_NS = _info.num_subcores
_NW = _NC * _NS

_G = 16
_TC_CHUNK = 1024


def _sc_gather(B, D):
  b_per_w = B // _NW
  n_groups = b_per_w // _G
  mesh = plsc.VectorSubcoreMesh(core_axis_name="c", subcore_axis_name="s")

  @functools.partial(
      pl.kernel,
      mesh=mesh,
      out_type=jax.ShapeDtypeStruct((B, D), jnp.float32),
      scratch_types=[
          pltpu.VMEM((b_per_w,), jnp.int32),
          pltpu.VMEM((b_per_w, D), jnp.float32),
          pltpu.SemaphoreType.DMA,
      ],
      cost_estimate=pl.CostEstimate(
          flops=0, bytes_accessed=600_000_000, transcendentals=0),
  )
  def gather_kernel(table_hbm, idx_hbm, out_hbm, idx_v, rows_v, sem):
    wid = lax.axis_index("s") * _NC + lax.axis_index("c")
    base = wid * b_per_w
    pltpu.sync_copy(idx_hbm.at[pl.ds(base, b_per_w)], idx_v)

    def fire(g, _):
      vec = idx_v[pl.ds(g * _G, _G)]
      for k in range(_G):
        pltpu.async_copy(table_hbm.at[vec[k]], rows_v.at[g * _G + k], sem)
      return 0

    lax.fori_loop(0, n_groups, fire, 0)

    def drain(g, _):
      for k in range(_G):
        pltpu.make_async_copy(table_hbm.at[0], rows_v.at[0], sem).wait()
      return 0

    lax.fori_loop(0, n_groups, drain, 0)
    pltpu.sync_copy(rows_v, out_hbm.at[pl.ds(base, b_per_w)])

  return gather_kernel


def _tc_gather(B, D):
  n_chunks = B // _TC_CHUNK

  def body(idx_hbm, table_hbm, out_hbm, vmem, smem_idx, sem_i, sem_r):
    def chunk(c, _):
      pltpu.make_async_copy(idx_hbm.at[pl.ds(c * _TC_CHUNK, _TC_CHUNK)],
                            smem_idx, sem_i).start()
      pltpu.make_async_copy(idx_hbm.at[pl.ds(c * _TC_CHUNK, _TC_CHUNK)],
                            smem_idx, sem_i).wait()

      def row(i, _):
        r = smem_idx[i]
        pltpu.make_async_copy(table_hbm.at[r], vmem.at[c * _TC_CHUNK + i],
                              sem_r).start()
        return 0

      lax.fori_loop(0, _TC_CHUNK, row, 0)
      return 0

    lax.fori_loop(0, n_chunks, chunk, 0)

    def drain(i, _):
      pltpu.make_async_copy(table_hbm.at[0], vmem.at[0], sem_r).wait()
      return 0

    lax.fori_loop(0, B, drain, 0)
    pltpu.make_async_copy(vmem, out_hbm, sem_i).start()
    pltpu.make_async_copy(vmem, out_hbm, sem_i).wait()

  return pl.pallas_call(
      body,
      out_shape=jax.ShapeDtypeStruct((B, D), jnp.float32),
      in_specs=[
          pl.BlockSpec(memory_space=pl.ANY),
          pl.BlockSpec(memory_space=pl.ANY),
      ],
      out_specs=pl.BlockSpec(memory_space=pl.ANY),
      scratch_shapes=[
          pltpu.VMEM((B, D), jnp.float32),
          pltpu.SMEM((_TC_CHUNK,), jnp.int32),
          pltpu.SemaphoreType.DMA,
          pltpu.SemaphoreType.DMA,
      ],
  )


@jax.jit
def kernel(users, U_g):
  flat = users.reshape(-1).astype(jnp.int32)
  B, D = flat.shape[0], U_g.shape[1]
  b_sc = 10240
  out_sc = _sc_gather(b_sc, D)(U_g, flat[:b_sc])
  out_tc = _tc_gather(B - b_sc, D)(flat[b_sc:], U_g)
  out = jnp.concatenate([out_sc, out_tc], axis=0)
  return out.reshape(tuple(users.shape) + (D,))


# speedup vs baseline: 1.5208x; 1.2248x over previous
import functools

import jax
import jax.numpy as jnp
from jax import lax
from jax.experimental import pallas as pl
from jax.experimental.pallas import tpu as pltpu, tpu_sc as plsc

_info = plsc.get_sparse_core_info()
_NC = _info.num_cores
_NS = _info.num_subcores
_NW = _NC * _NS

_G = 16


def _make_gather(B, D):
  b_per_w = B // _NW
  n_groups = b_per_w // _G
  mesh = plsc.VectorSubcoreMesh(core_axis_name="c", subcore_axis_name="s")

  @functools.partial(
      pl.kernel,
      mesh=mesh,
      out_type=jax.ShapeDtypeStruct((B, D), jnp.float32),
      scratch_types=[
          pltpu.VMEM((b_per_w,), jnp.int32),
          pltpu.VMEM((b_per_w, D), jnp.float32),
          pltpu.SemaphoreType.DMA,
          pltpu.SemaphoreType.DMA,
      ],
  )
  def gather_kernel(table_hbm, idx_hbm, out_hbm, idx_v, rows_v, sem, wsem):
    wid = lax.axis_index("s") * _NC + lax.axis_index("c")
    base = wid * b_per_w
    pltpu.sync_copy(idx_hbm.at[pl.ds(base, b_per_w)], idx_v)

    def fire(g, _):
      vec = idx_v[pl.ds(g * _G, _G)]
      for k in range(_G):
        pltpu.async_copy(table_hbm.at[vec[k]], rows_v.at[g * _G + k], sem)
      return 0

    lax.fori_loop(0, n_groups, fire, 0)

    def drain(g, _):
      for k in range(_G):
        pltpu.make_async_copy(table_hbm.at[0], rows_v.at[0], sem).wait()
      return 0

    lax.fori_loop(0, n_groups, drain, 0)

    def wfire(g, _):
      for k in range(_G):
        j = g * _G + k
        pltpu.async_copy(rows_v.at[j], out_hbm.at[base + j], wsem)
      return 0

    lax.fori_loop(0, n_groups, wfire, 0)

    def wdrain(g, _):
      for k in range(_G):
        pltpu.make_async_copy(rows_v.at[0], out_hbm.at[base], wsem).wait()
      return 0

    lax.fori_loop(0, n_groups, wdrain, 0)

  return gather_kernel


@jax.jit
def kernel(users, U_g):
  flat = users.reshape(-1).astype(jnp.int32)
  out = _make_gather(flat.shape[0], U_g.shape[1])(U_g, flat)
  return out.reshape(tuple(users.shape) + (U_g.shape[1],))


# speedup vs baseline: 1.5231x; 1.0015x over previous
import functools

import jax
import jax.numpy as jnp
from jax import lax
from jax.experimental import pallas as pl
from jax.experimental.pallas import tpu as pltpu, tpu_sc as plsc

_info = plsc.get_sparse_core_info()
_NC = _info.num_cores
_NS = _info.num_subcores
_NW = _NC * _NS

_G = 16


def _make_gather(B, D):
  b_per_w = B // _NW
  n_groups = b_per_w // _G
  mesh = plsc.VectorSubcoreMesh(core_axis_name="c", subcore_axis_name="s")

  @functools.partial(
      pl.kernel,
      mesh=mesh,
      out_type=jax.ShapeDtypeStruct((B, D), jnp.float32),
      scratch_types=[
          pltpu.VMEM((b_per_w,), jnp.int32),
          pltpu.VMEM((b_per_w, D), jnp.float32),
          pltpu.SemaphoreType.DMA,
      ],
  )
  def gather_kernel(table_hbm, idx_hbm, out_hbm, idx_v, rows_v, sem):
    wid = lax.axis_index("s") * _NC + lax.axis_index("c")
    base = wid * b_per_w
    pltpu.sync_copy(idx_hbm.at[pl.ds(base, b_per_w)], idx_v)

    def fire(g, _):
      vec = idx_v[pl.ds(g * _G, _G)]
      for k in range(_G):
        pltpu.async_copy(table_hbm.at[vec[k]], rows_v.at[g * _G + k], sem)
      return 0

    lax.fori_loop(0, n_groups, fire, 0)

    def drain(g, _):
      for k in range(_G):
        pltpu.make_async_copy(table_hbm.at[0], rows_v.at[0], sem).wait()
      return 0

    lax.fori_loop(0, n_groups, drain, 0)
    pltpu.sync_copy(rows_v, out_hbm.at[pl.ds(base, b_per_w)])

  return gather_kernel


@jax.jit
def kernel(users, U_g):
  flat = users.reshape(-1).astype(jnp.int32)
  out = _make_gather(flat.shape[0], U_g.shape[1])(U_g, flat)
  return out.reshape(tuple(users.shape) + (U_g.shape[1],))
